# Initial kernel scaffold; baseline (speedup 1.0000x reference)
import jax
import jax.numpy as jnp
from jax.experimental import pallas as pl


def kernel(voxel_list, params):
    raise NotImplementedError("write your pallas kernel here")


# speedup vs baseline: 2.3620x; 2.3620x over previous
import jax
import jax.numpy as jnp
from jax.experimental import pallas as pl
from jax.experimental.pallas import tpu as pltpu

B = 64
NV = 15724
HID = 2048
NE0 = 4
NB = 4
CE = 256
CS = 77
OUT = CE * CS
H0 = HID // NE0
H1 = H0 // 2
NE1 = NE0 * 2

KT = 1024
NKT = (NV + KT - 1) // KT
KREM = NV - (NKT - 1) * KT

CS_T = 7
NJ = CS // CS_T
WT = CS_T * CE


def _dot(a, b):
    return jax.lax.dot_general(a, b, (((1,), (0,)), ((), ())),
                               preferred_element_type=jnp.float32)


def _ln_f(x, g, b):
    m = jnp.mean(x, -1, keepdims=True)
    v = jnp.mean((x - m) ** 2, -1, keepdims=True)
    return (x - m) * jax.lax.rsqrt(v + 1e-5) * g + b


def _ridge_body(x_ref, w_ref, b_ref, o_ref):
    k = pl.program_id(0)

    @pl.when(k == 0)
    def _():
        o_ref[...] = jnp.broadcast_to(b_ref[...], (B, HID))

    @pl.when(k < NKT - 1)
    def _():
        o_ref[...] += _dot(x_ref[...], w_ref[...])

    @pl.when(k == NKT - 1)
    def _():
        lane = jax.lax.broadcasted_iota(jnp.int32, (B, KT), 1)
        x = jnp.where(lane < KREM, x_ref[...], 0.0)
        sub = jax.lax.broadcasted_iota(jnp.int32, (KT, HID), 0)
        w = jnp.where(sub < KREM, w_ref[...], 0.0)
        o_ref[...] += _dot(x, w)


def _ridge(voxel, rw, rb):
    return pl.pallas_call(
        _ridge_body,
        grid=(NKT,),
        in_specs=[
            pl.BlockSpec((B, KT), lambda k: (0, k)),
            pl.BlockSpec((KT, HID), lambda k: (k, 0)),
            pl.BlockSpec((1, HID), lambda k: (0, 0)),
        ],
        out_specs=pl.BlockSpec((B, HID), lambda k: (0, 0)),
        out_shape=jax.ShapeDtypeStruct((B, HID), jnp.float32),
    )(voxel, rw, rb)


def _experts_body(vr_ref, g0w_ref, g0b_ref, *refs):
    eo_ref, e1_ref, lb_ref = refs[-3:]
    per = refs[:-3]
    vr = vr_ref[...]

    logits = _dot(vr, g0w_ref[...]) + g0b_ref[...]
    mx = jnp.max(logits, -1, keepdims=True)
    ex = jnp.exp(logits - mx)
    probs = ex / jnp.sum(ex, -1, keepdims=True)
    top = jnp.argmax(probs, axis=-1, keepdims=True)
    onehot = (jax.lax.broadcasted_iota(jnp.int32, (B, NE0), 1)
              == top).astype(jnp.float32)
    f = jnp.mean(onehot, axis=0, keepdims=True)
    P = jnp.mean(probs, axis=0, keepdims=True)
    lb_ref[...] = NE0 * jnp.sum(f * P, keepdims=True)

    for i in range(NE0):
        r = per[i * 24:(i + 1) * 24]
        x = vr[:, i * H0:(i + 1) * H0] * probs[:, i:i + 1]
        x = _dot(x, r[0][...]) + r[1][...]
        for t in range(NB):
            lng, lnb, fcw, fcb = (r[2 + 4 * t][...], r[3 + 4 * t][...],
                                  r[4 + 4 * t], r[5 + 4 * t])
            h = jax.nn.gelu(_ln_f(x, lng, lnb))
            x = x + _dot(h, fcw[...]) + fcb[...]
        eo_ref[i] = x
        lg = _dot(x, r[18][...]) + r[19][...]
        m1 = jnp.max(lg, -1, keepdims=True)
        e1 = jnp.exp(lg - m1)
        p1 = e1 / jnp.sum(e1, -1, keepdims=True)
        for j in range(2):
            xin = x[:, j * H1:(j + 1) * H1] * p1[:, j:j + 1]
            e1_ref[2 * i + j] = _dot(xin, r[20 + 2 * j][...]) + r[21 + 2 * j][...]


def _experts(vr, params):
    flat = [params["gate0"]["W"], params["gate0"]["b"].reshape(1, -1)]
    for i in range(NE0):
        e = params["experts0"][i]
        flat += [e["lin0"]["W"], e["lin0"]["b"].reshape(1, -1)]
        for blk in e["blocks"]:
            flat += [blk["ln"]["g"].reshape(1, -1), blk["ln"]["b"].reshape(1, -1),
                     blk["fc"]["W"], blk["fc"]["b"].reshape(1, -1)]
        g1 = params["gate1"][i]
        flat += [g1["W"], g1["b"].reshape(1, -1)]
        for j in range(2):
            e1 = params["experts1"][2 * i + j]
            flat += [e1["W"], e1["b"].reshape(1, -1)]
    return pl.pallas_call(
        _experts_body,
        out_shape=(
            jax.ShapeDtypeStruct((NE0, B, H0), jnp.float32),
            jax.ShapeDtypeStruct((NE1, B, H1), jnp.float32),
            jax.ShapeDtypeStruct((1, 1), jnp.float32),
        ),
    )(vr, *flat)


def _proj_mlp(y, cp):
    y = jax.nn.gelu(_ln_f(y, cp[0][...], cp[1][...]))
    y = jax.nn.gelu(_ln_f(_dot(y, cp[2][...]) + cp[3][...],
                          cp[4][...], cp[5][...]))
    y = jax.nn.gelu(_ln_f(_dot(y, cp[6][...]) + cp[7][...],
                          cp[8][...], cp[9][...]))
    return _dot(y, cp[10][...]) + cp[11][...]


def _rows_of(bo):
    return jnp.concatenate(
        [bo[:, c * CE:(c + 1) * CE] for c in range(CS_T)], axis=0)


def _cols_of(y):
    return jnp.concatenate(
        [y[c * B:(c + 1) * B, :] for c in range(CS_T)], axis=1)


def _dispatch_body(ne, merge, *refs):
    if merge:
        eo_ref, bt0_ref, ct0_ref = refs[0:3]
        rest = refs[3:]
    else:
        eo_ref = refs[0]
        rest = refs[1:]
    bt_ref, ct_ref = rest[-2:]
    wbs = rest[0:ne]
    bbs = rest[ne:2 * ne]
    cps = rest[2 * ne:2 * ne + 12 * ne]

    bt = None
    ct = None
    for i in range(ne):
        x = eo_ref[i]
        bo = _dot(x, wbs[i][...]) + bbs[i][...]
        bt = bo if bt is None else bt + bo
        y = _proj_mlp(_rows_of(bo), cps[12 * i:12 * (i + 1)])
        ct = y if ct is None else ct + y
    ct = _cols_of(ct)
    if merge:
        bt_ref[...] = (bt0_ref[...] + bt) * 0.5
        ct_ref[...] = (ct0_ref[...] + ct) * 0.5
    else:
        bt_ref[...] = bt
        ct_ref[...] = ct


def _dispatch(eo, bplist, cplist, ne, h, prev=None):
    flat = [bp["W"] for bp in bplist]
    flat += [bp["b"].reshape(1, -1) for bp in bplist]
    for cp in cplist:
        flat += [cp["ln0"]["g"].reshape(1, -1), cp["ln0"]["b"].reshape(1, -1),
                 cp["fc1"]["W"], cp["fc1"]["b"].reshape(1, -1),
                 cp["ln1"]["g"].reshape(1, -1), cp["ln1"]["b"].reshape(1, -1),
                 cp["fc2"]["W"], cp["fc2"]["b"].reshape(1, -1),
                 cp["ln2"]["g"].reshape(1, -1), cp["ln2"]["b"].reshape(1, -1),
                 cp["fc3"]["W"], cp["fc3"]["b"].reshape(1, -1)]
    in_specs = [pl.BlockSpec((ne, B, h), lambda j: (0, 0, 0))]
    args = [eo]
    merge = prev is not None
    if merge:
        in_specs += [pl.BlockSpec((B, WT), lambda j: (0, j))] * 2
        args += [prev[0], prev[1]]
    in_specs += [pl.BlockSpec((h, WT), lambda j: (0, j))] * ne
    in_specs += [pl.BlockSpec((1, WT), lambda j: (0, j))] * ne
    for _ in range(ne):
        in_specs += [
            pl.BlockSpec((1, CE), lambda j: (0, 0)),
            pl.BlockSpec((1, CE), lambda j: (0, 0)),
            pl.BlockSpec((CE, 2 * CE), lambda j: (0, 0)),
            pl.BlockSpec((1, 2 * CE), lambda j: (0, 0)),
            pl.BlockSpec((1, 2 * CE), lambda j: (0, 0)),
            pl.BlockSpec((1, 2 * CE), lambda j: (0, 0)),
            pl.BlockSpec((2 * CE, 2 * CE), lambda j: (0, 0)),
            pl.BlockSpec((1, 2 * CE), lambda j: (0, 0)),
            pl.BlockSpec((1, 2 * CE), lambda j: (0, 0)),
            pl.BlockSpec((1, 2 * CE), lambda j: (0, 0)),
            pl.BlockSpec((2 * CE, CE), lambda j: (0, 0)),
            pl.BlockSpec((1, CE), lambda j: (0, 0)),
        ]
    args += flat

    def body(*refs):
        _dispatch_body(ne, merge, *refs)

    return pl.pallas_call(
        body,
        grid=(NJ,),
        in_specs=in_specs,
        out_specs=(pl.BlockSpec((B, WT), lambda j: (0, j)),
                   pl.BlockSpec((B, WT), lambda j: (0, j))),
        out_shape=(jax.ShapeDtypeStruct((B, OUT), jnp.float32),
                   jax.ShapeDtypeStruct((B, OUT), jnp.float32)),
    )(*args)


def kernel(voxel_list, params):
    p = params
    vr = _ridge(voxel_list, p["ridge"]["W"], p["ridge"]["b"].reshape(1, -1))
    eo, e1, lb = _experts(vr, p)
    bt0, ct0 = _dispatch(eo, p["bproj0"], p["cproj0"], NE0, H0)
    bt, ct = _dispatch(e1, p["bproj1"], p["cproj1"], NE1, H1, prev=(bt0, ct0))
    return (bt.reshape(B, CS, CE), ct.reshape(B, CS, CE), lb.reshape(()))
